# baseline (speedup 1.0000x reference)
import dataclasses
import functools

import jax
import jax.numpy as jnp
from jax import lax
from jax.experimental import pallas as pl
from jax.experimental.pallas import tpu as pltpu
from jax.experimental.pallas import tpu_sc as plsc

NC = 2
NS = 16
L = 16

CH = 2048
BCH = 4096


def _sc_compiler_params():
    cp = pltpu.CompilerParams()
    if "needs_layout_passes" in pltpu.CompilerParams.__dataclass_fields__:
        cp = dataclasses.replace(cp, needs_layout_passes=False)
    if "use_tc_tiling_on_sc" in pltpu.CompilerParams.__dataclass_fields__:
        cp = dataclasses.replace(cp, use_tc_tiling_on_sc=False)
    return cp


@functools.partial(jax.jit, static_argnums=(3, 4))
def _sc_build_table_native(map0, map1, subj, RPC0, RPC1):
    M, C, H0, W0 = map0.shape
    H1, W1 = map1.shape[2], map1.shape[3]
    T0, T1 = H0 * W0, H1 * W1
    NW = NC * NS
    n0 = H0 // NW // RPC0
    n1 = H1 // NW // RPC1
    mesh = plsc.VectorSubcoreMesh(core_axis_name="c", subcore_axis_name="s", num_cores=NC, num_subcores=NS)

    @functools.partial(
        pl.kernel,
        out_type=jax.ShapeDtypeStruct(((T0 + T1) // 2, 8), jnp.float32),
        mesh=mesh,
        compiler_params=_sc_compiler_params(),
        scratch_types=[
            [pltpu.VMEM((3, RPC0, W0), jnp.float32) for _ in range(2)],
            [pltpu.VMEM((3, RPC1, W1), jnp.float32) for _ in range(2)],
            [pltpu.VMEM((RPC0 * W0 // 2, 8), jnp.float32) for _ in range(2)],
            pltpu.VMEM((16,), jnp.int32),
            [pltpu.SemaphoreType.DMA for _ in range(4)],
        ],
    )
    def kern(m0_hbm, m1_hbm, subj_hbm, out_hbm, sb0, sb1, obufs, sid_s, sems):
        wid = lax.axis_index("s") * NC + lax.axis_index("c")
        lane = lax.iota(jnp.int32, L)
        pltpu.sync_copy(subj_hbm, sid_s.at[pl.ds(0, 1)])
        sid = sid_s[pl.ds(0, L)][0]

        def run_map(src_hbm, sbufs, W, rpc, n_chunks, out_row_base):
            bch = rpc * W
            row0 = wid * n_chunks * rpc

            def fire_in(g, b):
                for ch in range(3):
                    pltpu.async_copy(
                        src_hbm.at[sid, ch, pl.ds(row0 + g * rpc, rpc), :],
                        sbufs[b].at[ch], sems[b])

            def wait_in(b):
                for ch in range(3):
                    pltpu.make_async_copy(
                        src_hbm.at[0, 0, pl.ds(0, rpc), :],
                        sbufs[b].at[ch], sems[b]).wait()

            def out_dst(g):
                off = (out_row_base + (row0 + g * rpc) * W) // 2
                return out_hbm.at[pl.ds(off, bch // 2)]

            def wait_out(b):
                pltpu.make_async_copy(
                    obufs[b].at[pl.ds(0, bch // 2)], out_dst(0), sems[2 + b]).wait()

            fire_in(0, 0)
            fire_in(1, 1)

            @pl.loop(0, n_chunks, step=2)
            def _(t):
                for b in range(2):
                    g = t + b
                    wait_in(b)

                    @pl.when(g >= 2)
                    def _():
                        wait_out(b)

                    for r in range(rpc):
                        @pl.loop(0, W // L)
                        def _(j):
                            pos = r * W + j * L + lane
                            row = pos >> 1
                            col = (pos & 1) << 2
                            for ch in range(3):
                                v = sbufs[b].at[ch].at[r][pl.ds(j * L, L)]
                                plsc.store_scatter(
                                    obufs[b].at[pl.ds(0, bch // 2)],
                                    [row, col + ch], v)

                    pltpu.async_copy(obufs[b].at[pl.ds(0, bch // 2)],
                                     out_dst(g), sems[2 + b])

                    @pl.when(g + 2 < n_chunks)
                    def _():
                        fire_in(g + 2, b)

            for b in range(2):
                wait_out(b)

        run_map(m0_hbm, sb0, W0, RPC0, n0, 0)
        run_map(m1_hbm, sb1, W1, RPC1, n1, T0)

    return kern(map0, map1, subj)


@functools.partial(jax.jit, static_argnums=(2, 3))
def _sc_build_table(m0sel, m1sel, T0, T1):
    NW = NC * NS
    n0 = T0 // NW // BCH
    n1 = T1 // NW // BCH
    mesh = plsc.VectorSubcoreMesh(core_axis_name="c", subcore_axis_name="s", num_cores=NC, num_subcores=NS)

    @functools.partial(
        pl.kernel,
        out_type=jax.ShapeDtypeStruct(((T0 + T1) // 2, 8), jnp.float32),
        mesh=mesh,
        compiler_params=_sc_compiler_params(),
        scratch_types=[
            [pltpu.VMEM((4, BCH), jnp.float32) for _ in range(2)],
            [pltpu.VMEM((BCH // 2, 8), jnp.float32) for _ in range(2)],
            [pltpu.SemaphoreType.DMA for _ in range(4)],
        ],
    )
    def kern(m0_hbm, m1_hbm, out_hbm, sbufs, obufs, sems):
        wid = lax.axis_index("s") * NC + lax.axis_index("c")
        lane = lax.iota(jnp.int32, L)

        def run_map(src_hbm, plane_words, n_chunks, out_row_base):
            per = n_chunks * BCH
            base = wid * per

            def fire_in(g, b):
                for ch in range(3):
                    pltpu.async_copy(
                        src_hbm.at[pl.ds(base + ch * plane_words + g * BCH, BCH)],
                        sbufs[b].at[ch], sems[b])

            def wait_in(b):
                for ch in range(3):
                    pltpu.make_async_copy(
                        src_hbm.at[pl.ds(0, BCH)], sbufs[b].at[ch], sems[b]).wait()

            def out_dst(g):
                off = (out_row_base + wid * per + g * BCH) // 2
                return out_hbm.at[pl.ds(off, BCH // 2)]

            fire_in(0, 0)
            fire_in(1, 1)

            @pl.loop(0, n_chunks, step=2)
            def _(t):
                for b in range(2):
                    g = t + b
                    wait_in(b)

                    @pl.when(g >= 2)
                    def _():
                        pltpu.make_async_copy(obufs[b], out_dst(0), sems[2 + b]).wait()

                    @pl.loop(0, BCH // L, step=4)
                    def _(j):
                        for u in range(4):
                            pos = (j + u) * L + lane
                            row = pos >> 1
                            col = (pos & 1) << 2
                            for ch in range(3):
                                v = sbufs[b].at[ch][pl.ds((j + u) * L, L)]
                                plsc.store_scatter(obufs[b], [row, col + ch], v)

                    pltpu.async_copy(obufs[b], out_dst(g), sems[2 + b])

                    @pl.when(g + 2 < n_chunks)
                    def _():
                        fire_in(g + 2, b)

            for b in range(2):
                pltpu.make_async_copy(obufs[b], out_dst(0), sems[2 + b]).wait()

        run_map(m0_hbm, T0, n0, 0)
        run_map(m1_hbm, T1, n1, T0)

    return kern(m0sel, m1sel)


@functools.partial(jax.jit, static_argnums=(3, 4, 5, 6))
def _sc_sample(table, uv_flat, idc_flat, H0, W0, H1, W1):
    P = idc_flat.shape[0]
    NW = NC * NS
    per_w = P // NW
    iters = per_w // CH
    base1 = H0 * W0
    mesh = plsc.VectorSubcoreMesh(core_axis_name="c", subcore_axis_name="s", num_cores=NC, num_subcores=NS)

    @functools.partial(
        pl.kernel,
        out_type=jax.ShapeDtypeStruct((3 * P,), jnp.float32),
        mesh=mesh,
        compiler_params=_sc_compiler_params(),
        scratch_types=[
            pltpu.VMEM((CH * 2,), jnp.float32),
            pltpu.VMEM((CH,), jnp.int32),
            [pltpu.VMEM((CH // 128, 128), jnp.int32) for _ in range(4)],
            [pltpu.VMEM((CH,), jnp.int32) for _ in range(4)],
            [pltpu.VMEM((CH,), jnp.float32) for _ in range(4)],
            [pltpu.VMEM((CH, 8), jnp.float32) for _ in range(4)],
            [pltpu.VMEM((CH,), jnp.float32) for _ in range(3)],
            pltpu.SemaphoreType.DMA,
        ],
    )
    def kern(t_hbm, uv_hbm, idc_hbm, out_hbm, uv_v, idc_v, r_vs, s_vs, w_vs, g_vs, o_vs, sem):
        wid = lax.axis_index("s") * NC + lax.axis_index("c")
        lane = lax.iota(jnp.int32, L)

        @pl.loop(0, iters)
        def _(t):
            base = wid * per_w + t * CH
            pltpu.sync_copy(uv_hbm.at[pl.ds(base * 2, CH * 2)], uv_v)
            pltpu.sync_copy(idc_hbm.at[pl.ds(base, CH)], idc_v)

            @pl.loop(0, CH // 128)
            def _(s):
                @pl.loop(0, 128 // L)
                def _(c):
                    off = s * 128 + c * L
                    u = plsc.load_gather(uv_v, [(off + lane) * 2])
                    v = plsc.load_gather(uv_v, [(off + lane) * 2 + 1])
                    m = idc_v[pl.ds(off, L)] == 0
                    halfw = jnp.where(m, (W0 - 1) * 0.5, (W1 - 1) * 0.5)
                    halfh = jnp.where(m, (H0 - 1) * 0.5, (H1 - 1) * 0.5)
                    maxx = jnp.where(m, float(W0 - 1), float(W1 - 1))
                    maxy = jnp.where(m, float(H0 - 1), float(H1 - 1))
                    wi = jnp.where(m, W0, W1)
                    bs = jnp.where(m, 0, base1)
                    x = jnp.clip((u + 1.0) * halfw, 0.0, maxx)
                    y = jnp.clip((v + 1.0) * halfh, 0.0, maxy)
                    xi = x.astype(jnp.int32)
                    yi = y.astype(jnp.int32)
                    wx1 = x - xi.astype(jnp.float32)
                    wy1 = y - yi.astype(jnp.float32)
                    x1 = jnp.minimum(xi + 1, maxx.astype(jnp.int32))
                    y1 = jnp.minimum(yi + 1, maxy.astype(jnp.int32))
                    r0 = bs + yi * wi
                    r1 = bs + y1 * wi
                    cs = pl.ds(c * L, L)
                    ls = pl.ds(off, L)
                    t00 = r0 + xi
                    t01 = r0 + x1
                    t10 = r1 + xi
                    t11 = r1 + x1
                    r_vs[0][s, cs] = t00 >> 1
                    r_vs[1][s, cs] = t01 >> 1
                    r_vs[2][s, cs] = t10 >> 1
                    r_vs[3][s, cs] = t11 >> 1
                    s_vs[0][ls] = (t00 & 1) << 2
                    s_vs[1][ls] = (t01 & 1) << 2
                    s_vs[2][ls] = (t10 & 1) << 2
                    s_vs[3][ls] = (t11 & 1) << 2
                    w_vs[0][ls] = (1.0 - wy1) * (1.0 - wx1)
                    w_vs[1][ls] = (1.0 - wy1) * wx1
                    w_vs[2][ls] = wy1 * (1.0 - wx1)
                    w_vs[3][ls] = wy1 * wx1

                for cn in range(4):
                    pltpu.async_copy(t_hbm.at[r_vs[cn].at[s]],
                                     g_vs[cn].at[pl.ds(s * 128, 128)], sem)

            for cn in range(4):
                for s in range(CH // 128):
                    pltpu.make_async_copy(
                        t_hbm.at[r_vs[cn].at[0]],
                        g_vs[cn].at[pl.ds(s * 128, 128)], sem).wait()

            @pl.loop(0, CH // L)
            def _(j):
                off = j * L
                pix = off + lane
                subs = [s_vs[cn][pl.ds(off, L)] for cn in range(4)]
                ws = [w_vs[cn][pl.ds(off, L)] for cn in range(4)]
                for ch in range(3):
                    acc = plsc.load_gather(g_vs[0], [pix, subs[0] + ch]) * ws[0]
                    for cn in range(1, 4):
                        acc = acc + plsc.load_gather(g_vs[cn], [pix, subs[cn] + ch]) * ws[cn]
                    o_vs[ch][pl.ds(off, L)] = acc

            for ch in range(3):
                pltpu.sync_copy(o_vs[ch], out_hbm.at[pl.ds(ch * P + base, CH)])

    return kern(table, uv_flat, idc_flat)


def kernel(uv_coords, uv_idcs, subject_id, map0, map1):
    N, H, W, _ = uv_coords.shape
    M, C, H0, W0 = map0.shape
    H1, W1 = map1.shape[2], map1.shape[3]
    T0, T1 = H0 * W0, H1 * W1
    table = _sc_build_table_native(map0, map1, subject_id.astype(jnp.int32), 4, 2)
    out = _sc_sample(table, uv_coords.reshape(-1), uv_idcs.reshape(-1).astype(jnp.int32),
                     H0, W0, H1, W1)
    return out.reshape(N, C, H, W)


# speedup vs baseline: 1.1357x; 1.1357x over previous
import dataclasses
import functools

import jax
import jax.numpy as jnp
from jax import lax
from jax.experimental import pallas as pl
from jax.experimental.pallas import tpu as pltpu
from jax.experimental.pallas import tpu_sc as plsc

NC = 2
NS = 16
L = 16

CH = 2048
BCH = 8192


def _sc_compiler_params():
    cp = pltpu.CompilerParams()
    if "needs_layout_passes" in pltpu.CompilerParams.__dataclass_fields__:
        cp = dataclasses.replace(cp, needs_layout_passes=False)
    if "use_tc_tiling_on_sc" in pltpu.CompilerParams.__dataclass_fields__:
        cp = dataclasses.replace(cp, use_tc_tiling_on_sc=False)
    return cp


@functools.partial(jax.jit, static_argnums=(3, 4))
def _sc_build_table_native(map0, map1, subj, RPC0, RPC1):
    M, C, H0, W0 = map0.shape
    H1, W1 = map1.shape[2], map1.shape[3]
    T0, T1 = H0 * W0, H1 * W1
    NW = NC * NS
    n0 = H0 // NW // RPC0
    n1 = H1 // NW // RPC1
    mesh = plsc.VectorSubcoreMesh(core_axis_name="c", subcore_axis_name="s", num_cores=NC, num_subcores=NS)

    @functools.partial(
        pl.kernel,
        out_type=jax.ShapeDtypeStruct(((T0 + T1) // 2, 8), jnp.float32),
        mesh=mesh,
        compiler_params=_sc_compiler_params(),
        scratch_types=[
            [pltpu.VMEM((3, RPC0, W0), jnp.float32) for _ in range(2)],
            [pltpu.VMEM((3, RPC1, W1), jnp.float32) for _ in range(2)],
            [pltpu.VMEM((RPC0 * W0 // 2, 8), jnp.float32) for _ in range(2)],
            pltpu.VMEM((16,), jnp.int32),
            [pltpu.SemaphoreType.DMA for _ in range(4)],
        ],
    )
    def kern(m0_hbm, m1_hbm, subj_hbm, out_hbm, sb0, sb1, obufs, sid_s, sems):
        wid = lax.axis_index("s") * NC + lax.axis_index("c")
        lane = lax.iota(jnp.int32, L)
        pltpu.sync_copy(subj_hbm, sid_s.at[pl.ds(0, 1)])
        sid = sid_s[pl.ds(0, L)][0]

        def run_map(src_hbm, sbufs, W, rpc, n_chunks, out_row_base):
            bch = rpc * W
            row0 = wid * n_chunks * rpc

            def fire_in(g, b):
                for ch in range(3):
                    pltpu.async_copy(
                        src_hbm.at[sid, ch, pl.ds(row0 + g * rpc, rpc), :],
                        sbufs[b].at[ch], sems[b])

            def wait_in(b):
                for ch in range(3):
                    pltpu.make_async_copy(
                        src_hbm.at[0, 0, pl.ds(0, rpc), :],
                        sbufs[b].at[ch], sems[b]).wait()

            def out_dst(g):
                off = (out_row_base + (row0 + g * rpc) * W) // 2
                return out_hbm.at[pl.ds(off, bch // 2)]

            def wait_out(b):
                pltpu.make_async_copy(
                    obufs[b].at[pl.ds(0, bch // 2)], out_dst(0), sems[2 + b]).wait()

            fire_in(0, 0)
            fire_in(1, 1)

            @pl.loop(0, n_chunks, step=2)
            def _(t):
                for b in range(2):
                    g = t + b
                    wait_in(b)

                    @pl.when(g >= 2)
                    def _():
                        wait_out(b)

                    for r in range(rpc):
                        @pl.loop(0, W // L)
                        def _(j):
                            pos = r * W + j * L + lane
                            row = pos >> 1
                            col = (pos & 1) << 2
                            for ch in range(3):
                                v = sbufs[b].at[ch].at[r][pl.ds(j * L, L)]
                                plsc.store_scatter(
                                    obufs[b].at[pl.ds(0, bch // 2)],
                                    [row, col + ch], v)

                    pltpu.async_copy(obufs[b].at[pl.ds(0, bch // 2)],
                                     out_dst(g), sems[2 + b])

                    @pl.when(g + 2 < n_chunks)
                    def _():
                        fire_in(g + 2, b)

            for b in range(2):
                wait_out(b)

        run_map(m0_hbm, sb0, W0, RPC0, n0, 0)
        run_map(m1_hbm, sb1, W1, RPC1, n1, T0)

    return kern(map0, map1, subj)


@functools.partial(jax.jit, static_argnums=(2, 3))
def _sc_build_table(m0sel, m1sel, T0, T1):
    NW = NC * NS
    n0 = T0 // NW // BCH
    n1 = T1 // NW // BCH
    mesh = plsc.VectorSubcoreMesh(core_axis_name="c", subcore_axis_name="s", num_cores=NC, num_subcores=NS)

    @functools.partial(
        pl.kernel,
        out_type=jax.ShapeDtypeStruct(((T0 + T1) // 2, 8), jnp.float32),
        mesh=mesh,
        compiler_params=_sc_compiler_params(),
        scratch_types=[
            [pltpu.VMEM((3, BCH), jnp.float32) for _ in range(2)],
            [pltpu.VMEM((BCH // 2, 8), jnp.float32) for _ in range(2)],
            [pltpu.SemaphoreType.DMA for _ in range(4)],
        ],
    )
    def kern(m0_hbm, m1_hbm, out_hbm, sbufs, obufs, sems):
        wid = lax.axis_index("s") * NC + lax.axis_index("c")
        lane = lax.iota(jnp.int32, L)

        def run_map(src_hbm, plane_words, n_chunks, out_row_base):
            per = n_chunks * BCH
            base = wid * per

            def fire_in(g, b):
                for ch in range(3):
                    pltpu.async_copy(
                        src_hbm.at[pl.ds(base + ch * plane_words + g * BCH, BCH)],
                        sbufs[b].at[ch], sems[b])

            def wait_in(b):
                for ch in range(3):
                    pltpu.make_async_copy(
                        src_hbm.at[pl.ds(0, BCH)], sbufs[b].at[ch], sems[b]).wait()

            def out_dst(g):
                off = (out_row_base + wid * per + g * BCH) // 2
                return out_hbm.at[pl.ds(off, BCH // 2)]

            fire_in(0, 0)
            fire_in(1, 1)

            @pl.loop(0, n_chunks, step=2)
            def _(t):
                for b in range(2):
                    g = t + b
                    wait_in(b)

                    @pl.when(g >= 2)
                    def _():
                        pltpu.make_async_copy(obufs[b], out_dst(0), sems[2 + b]).wait()

                    @pl.loop(0, BCH // L, step=4)
                    def _(j):
                        for u in range(4):
                            pos = (j + u) * L + lane
                            row = pos >> 1
                            col = (pos & 1) << 2
                            for ch in range(3):
                                v = sbufs[b].at[ch][pl.ds((j + u) * L, L)]
                                plsc.store_scatter(obufs[b], [row, col + ch], v)

                    pltpu.async_copy(obufs[b], out_dst(g), sems[2 + b])

                    @pl.when(g + 2 < n_chunks)
                    def _():
                        fire_in(g + 2, b)

            for b in range(2):
                pltpu.make_async_copy(obufs[b], out_dst(0), sems[2 + b]).wait()

        run_map(m0_hbm, T0, n0, 0)
        run_map(m1_hbm, T1, n1, T0)

    return kern(m0sel, m1sel)


@functools.partial(jax.jit, static_argnums=(3, 4, 5, 6))
def _sc_sample(table, uv_flat, idc_flat, H0, W0, H1, W1):
    P = idc_flat.shape[0]
    NW = NC * NS
    per_w = P // NW
    iters = per_w // CH
    base1 = H0 * W0
    mesh = plsc.VectorSubcoreMesh(core_axis_name="c", subcore_axis_name="s", num_cores=NC, num_subcores=NS)

    @functools.partial(
        pl.kernel,
        out_type=jax.ShapeDtypeStruct((3 * P,), jnp.float32),
        mesh=mesh,
        compiler_params=_sc_compiler_params(),
        scratch_types=[
            pltpu.VMEM((CH * 2,), jnp.float32),
            pltpu.VMEM((CH,), jnp.int32),
            [pltpu.VMEM((CH // 128, 128), jnp.int32) for _ in range(4)],
            [pltpu.VMEM((CH,), jnp.int32) for _ in range(4)],
            [pltpu.VMEM((CH,), jnp.float32) for _ in range(4)],
            [pltpu.VMEM((CH, 8), jnp.float32) for _ in range(4)],
            [pltpu.VMEM((CH,), jnp.float32) for _ in range(3)],
            pltpu.SemaphoreType.DMA,
        ],
    )
    def kern(t_hbm, uv_hbm, idc_hbm, out_hbm, uv_v, idc_v, r_vs, s_vs, w_vs, g_vs, o_vs, sem):
        wid = lax.axis_index("s") * NC + lax.axis_index("c")
        lane = lax.iota(jnp.int32, L)

        @pl.loop(0, iters)
        def _(t):
            base = wid * per_w + t * CH
            pltpu.sync_copy(uv_hbm.at[pl.ds(base * 2, CH * 2)], uv_v)
            pltpu.sync_copy(idc_hbm.at[pl.ds(base, CH)], idc_v)

            @pl.loop(0, CH // 128)
            def _(s):
                @pl.loop(0, 128 // L)
                def _(c):
                    off = s * 128 + c * L
                    u = plsc.load_gather(uv_v, [(off + lane) * 2])
                    v = plsc.load_gather(uv_v, [(off + lane) * 2 + 1])
                    m = idc_v[pl.ds(off, L)] == 0
                    halfw = jnp.where(m, (W0 - 1) * 0.5, (W1 - 1) * 0.5)
                    halfh = jnp.where(m, (H0 - 1) * 0.5, (H1 - 1) * 0.5)
                    maxx = jnp.where(m, float(W0 - 1), float(W1 - 1))
                    maxy = jnp.where(m, float(H0 - 1), float(H1 - 1))
                    wi = jnp.where(m, W0, W1)
                    bs = jnp.where(m, 0, base1)
                    x = jnp.clip((u + 1.0) * halfw, 0.0, maxx)
                    y = jnp.clip((v + 1.0) * halfh, 0.0, maxy)
                    xi = x.astype(jnp.int32)
                    yi = y.astype(jnp.int32)
                    wx1 = x - xi.astype(jnp.float32)
                    wy1 = y - yi.astype(jnp.float32)
                    x1 = jnp.minimum(xi + 1, maxx.astype(jnp.int32))
                    y1 = jnp.minimum(yi + 1, maxy.astype(jnp.int32))
                    r0 = bs + yi * wi
                    r1 = bs + y1 * wi
                    cs = pl.ds(c * L, L)
                    ls = pl.ds(off, L)
                    t00 = r0 + xi
                    t01 = r0 + x1
                    t10 = r1 + xi
                    t11 = r1 + x1
                    r_vs[0][s, cs] = t00 >> 1
                    r_vs[1][s, cs] = t01 >> 1
                    r_vs[2][s, cs] = t10 >> 1
                    r_vs[3][s, cs] = t11 >> 1
                    s_vs[0][ls] = (t00 & 1) << 2
                    s_vs[1][ls] = (t01 & 1) << 2
                    s_vs[2][ls] = (t10 & 1) << 2
                    s_vs[3][ls] = (t11 & 1) << 2
                    w_vs[0][ls] = (1.0 - wy1) * (1.0 - wx1)
                    w_vs[1][ls] = (1.0 - wy1) * wx1
                    w_vs[2][ls] = wy1 * (1.0 - wx1)
                    w_vs[3][ls] = wy1 * wx1

                for cn in range(4):
                    pltpu.async_copy(t_hbm.at[r_vs[cn].at[s]],
                                     g_vs[cn].at[pl.ds(s * 128, 128)], sem)

            for cn in range(4):
                for s in range(CH // 128):
                    pltpu.make_async_copy(
                        t_hbm.at[r_vs[cn].at[0]],
                        g_vs[cn].at[pl.ds(s * 128, 128)], sem).wait()

            @pl.loop(0, CH // L)
            def _(j):
                off = j * L
                pix = off + lane
                subs = [s_vs[cn][pl.ds(off, L)] for cn in range(4)]
                ws = [w_vs[cn][pl.ds(off, L)] for cn in range(4)]
                for ch in range(3):
                    acc = plsc.load_gather(g_vs[0], [pix, subs[0] + ch]) * ws[0]
                    for cn in range(1, 4):
                        acc = acc + plsc.load_gather(g_vs[cn], [pix, subs[cn] + ch]) * ws[cn]
                    o_vs[ch][pl.ds(off, L)] = acc

            for ch in range(3):
                pltpu.sync_copy(o_vs[ch], out_hbm.at[pl.ds(ch * P + base, CH)])

    return kern(table, uv_flat, idc_flat)


def kernel(uv_coords, uv_idcs, subject_id, map0, map1):
    N, H, W, _ = uv_coords.shape
    M, C, H0, W0 = map0.shape
    H1, W1 = map1.shape[2], map1.shape[3]
    T0, T1 = H0 * W0, H1 * W1
    sid = subject_id[0]
    table = _sc_build_table(map0[sid].reshape(-1), map1[sid].reshape(-1), T0, T1)
    out = _sc_sample(table, uv_coords.reshape(-1), uv_idcs.reshape(-1).astype(jnp.int32),
                     H0, W0, H1, W1)
    return out.reshape(N, C, H, W)


# speedup vs baseline: 1.1527x; 1.0150x over previous
import dataclasses
import functools

import jax
import jax.numpy as jnp
from jax import lax
from jax.experimental import pallas as pl
from jax.experimental.pallas import tpu as pltpu
from jax.experimental.pallas import tpu_sc as plsc

NC = 2
NS = 16
L = 16

CH = 2048
BCH = 8192


def _sc_compiler_params():
    cp = pltpu.CompilerParams()
    if "needs_layout_passes" in pltpu.CompilerParams.__dataclass_fields__:
        cp = dataclasses.replace(cp, needs_layout_passes=False)
    if "use_tc_tiling_on_sc" in pltpu.CompilerParams.__dataclass_fields__:
        cp = dataclasses.replace(cp, use_tc_tiling_on_sc=False)
    return cp


@functools.partial(jax.jit, static_argnums=(3, 4))
def _sc_build_table_native(map0, map1, subj, RPC0, RPC1):
    M, C, H0, W0 = map0.shape
    H1, W1 = map1.shape[2], map1.shape[3]
    T0, T1 = H0 * W0, H1 * W1
    NW = NC * NS
    n0 = H0 // NW // RPC0
    n1 = H1 // NW // RPC1
    mesh = plsc.VectorSubcoreMesh(core_axis_name="c", subcore_axis_name="s", num_cores=NC, num_subcores=NS)

    @functools.partial(
        pl.kernel,
        out_type=jax.ShapeDtypeStruct(((T0 + T1) // 2, 8), jnp.float32),
        mesh=mesh,
        compiler_params=_sc_compiler_params(),
        scratch_types=[
            [pltpu.VMEM((3, RPC0, W0), jnp.float32) for _ in range(2)],
            [pltpu.VMEM((3, RPC1, W1), jnp.float32) for _ in range(2)],
            [pltpu.VMEM((RPC0 * W0 // 2, 8), jnp.float32) for _ in range(2)],
            pltpu.VMEM((16,), jnp.int32),
            [pltpu.SemaphoreType.DMA for _ in range(4)],
        ],
    )
    def kern(m0_hbm, m1_hbm, subj_hbm, out_hbm, sb0, sb1, obufs, sid_s, sems):
        wid = lax.axis_index("s") * NC + lax.axis_index("c")
        lane = lax.iota(jnp.int32, L)
        pltpu.sync_copy(subj_hbm, sid_s.at[pl.ds(0, 1)])
        sid = sid_s[pl.ds(0, L)][0]

        def run_map(src_hbm, sbufs, W, rpc, n_chunks, out_row_base):
            bch = rpc * W
            row0 = wid * n_chunks * rpc

            def fire_in(g, b):
                for ch in range(3):
                    pltpu.async_copy(
                        src_hbm.at[sid, ch, pl.ds(row0 + g * rpc, rpc), :],
                        sbufs[b].at[ch], sems[b])

            def wait_in(b):
                for ch in range(3):
                    pltpu.make_async_copy(
                        src_hbm.at[0, 0, pl.ds(0, rpc), :],
                        sbufs[b].at[ch], sems[b]).wait()

            def out_dst(g):
                off = (out_row_base + (row0 + g * rpc) * W) // 2
                return out_hbm.at[pl.ds(off, bch // 2)]

            def wait_out(b):
                pltpu.make_async_copy(
                    obufs[b].at[pl.ds(0, bch // 2)], out_dst(0), sems[2 + b]).wait()

            fire_in(0, 0)
            fire_in(1, 1)

            @pl.loop(0, n_chunks, step=2)
            def _(t):
                for b in range(2):
                    g = t + b
                    wait_in(b)

                    @pl.when(g >= 2)
                    def _():
                        wait_out(b)

                    for r in range(rpc):
                        @pl.loop(0, W // L)
                        def _(j):
                            pos = r * W + j * L + lane
                            row = pos >> 1
                            col = (pos & 1) << 2
                            for ch in range(3):
                                v = sbufs[b].at[ch].at[r][pl.ds(j * L, L)]
                                plsc.store_scatter(
                                    obufs[b].at[pl.ds(0, bch // 2)],
                                    [row, col + ch], v)

                    pltpu.async_copy(obufs[b].at[pl.ds(0, bch // 2)],
                                     out_dst(g), sems[2 + b])

                    @pl.when(g + 2 < n_chunks)
                    def _():
                        fire_in(g + 2, b)

            for b in range(2):
                wait_out(b)

        run_map(m0_hbm, sb0, W0, RPC0, n0, 0)
        run_map(m1_hbm, sb1, W1, RPC1, n1, T0)

    return kern(map0, map1, subj)


@functools.partial(jax.jit, static_argnums=(2, 3))
def _sc_build_table(m0sel, m1sel, T0, T1):
    NW = NC * NS
    n0 = T0 // NW // BCH
    n1 = T1 // NW // BCH
    mesh = plsc.VectorSubcoreMesh(core_axis_name="c", subcore_axis_name="s", num_cores=NC, num_subcores=NS)

    @functools.partial(
        pl.kernel,
        out_type=jax.ShapeDtypeStruct(((T0 + T1) // 2, 8), jnp.float32),
        mesh=mesh,
        compiler_params=_sc_compiler_params(),
        scratch_types=[
            [pltpu.VMEM((3, BCH), jnp.float32) for _ in range(2)],
            [pltpu.VMEM((BCH // 2, 8), jnp.float32) for _ in range(2)],
            [pltpu.SemaphoreType.DMA for _ in range(4)],
        ],
    )
    def kern(m0_hbm, m1_hbm, out_hbm, sbufs, obufs, sems):
        wid = lax.axis_index("s") * NC + lax.axis_index("c")
        lane = lax.iota(jnp.int32, L)

        def run_map(src_hbm, plane_words, n_chunks, out_row_base):
            per = n_chunks * BCH
            base = wid * per

            def fire_in(g, b):
                for ch in range(3):
                    pltpu.async_copy(
                        src_hbm.at[pl.ds(base + ch * plane_words + g * BCH, BCH)],
                        sbufs[b].at[ch], sems[b])

            def wait_in(b):
                for ch in range(3):
                    pltpu.make_async_copy(
                        src_hbm.at[pl.ds(0, BCH)], sbufs[b].at[ch], sems[b]).wait()

            def out_dst(g):
                off = (out_row_base + wid * per + g * BCH) // 2
                return out_hbm.at[pl.ds(off, BCH // 2)]

            fire_in(0, 0)
            fire_in(1, 1)

            @pl.loop(0, n_chunks, step=2)
            def _(t):
                for b in range(2):
                    g = t + b
                    wait_in(b)

                    @pl.when(g >= 2)
                    def _():
                        pltpu.make_async_copy(obufs[b], out_dst(0), sems[2 + b]).wait()

                    @pl.loop(0, BCH // L, step=4)
                    def _(j):
                        for u in range(4):
                            pos = (j + u) * L + lane
                            row = pos >> 1
                            col = (pos & 1) << 2
                            for ch in range(3):
                                v = sbufs[b].at[ch][pl.ds((j + u) * L, L)]
                                plsc.store_scatter(obufs[b], [row, col + ch], v)

                    pltpu.async_copy(obufs[b], out_dst(g), sems[2 + b])

                    @pl.when(g + 2 < n_chunks)
                    def _():
                        fire_in(g + 2, b)

            for b in range(2):
                pltpu.make_async_copy(obufs[b], out_dst(0), sems[2 + b]).wait()

        run_map(m0_hbm, T0, n0, 0)
        run_map(m1_hbm, T1, n1, T0)

    return kern(m0sel, m1sel)


@functools.partial(jax.jit, static_argnums=(3, 4, 5, 6))
def _sc_sample(table, uv_flat, idc_flat, H0, W0, H1, W1):
    P = idc_flat.shape[0]
    NW = NC * NS
    per_w = P // NW
    iters = per_w // CH
    base1 = H0 * W0
    mesh = plsc.VectorSubcoreMesh(core_axis_name="c", subcore_axis_name="s", num_cores=NC, num_subcores=NS)

    @functools.partial(
        pl.kernel,
        out_type=jax.ShapeDtypeStruct((3 * P,), jnp.float32),
        mesh=mesh,
        compiler_params=_sc_compiler_params(),
        scratch_types=[
            [pltpu.VMEM((CH * 2,), jnp.float32) for _ in range(2)],
            [pltpu.VMEM((CH,), jnp.int32) for _ in range(2)],
            [pltpu.VMEM((CH // 128, 128), jnp.int32) for _ in range(4)],
            [pltpu.VMEM((CH,), jnp.int32) for _ in range(4)],
            [pltpu.VMEM((CH,), jnp.float32) for _ in range(4)],
            [pltpu.VMEM((CH, 8), jnp.float32) for _ in range(4)],
            [[pltpu.VMEM((CH,), jnp.float32) for _ in range(3)] for _ in range(2)],
            pltpu.SemaphoreType.DMA,
            [pltpu.SemaphoreType.DMA for _ in range(2)],
            [pltpu.SemaphoreType.DMA for _ in range(2)],
        ],
    )
    def kern(t_hbm, uv_hbm, idc_hbm, out_hbm, uv_vs, idc_vs, r_vs, s_vs, w_vs, g_vs,
             o_vs2, sem, sem_in, sem_out):
        wid = lax.axis_index("s") * NC + lax.axis_index("c")
        lane = lax.iota(jnp.int32, L)

        def fire_stage(g, b):
            base = wid * per_w + g * CH
            pltpu.async_copy(uv_hbm.at[pl.ds(base * 2, CH * 2)], uv_vs[b], sem_in[b])
            pltpu.async_copy(idc_hbm.at[pl.ds(base, CH)], idc_vs[b], sem_in[b])

        def wait_stage(b):
            pltpu.make_async_copy(uv_hbm.at[pl.ds(0, CH * 2)], uv_vs[b], sem_in[b]).wait()
            pltpu.make_async_copy(idc_hbm.at[pl.ds(0, CH)], idc_vs[b], sem_in[b]).wait()

        def wait_out(b):
            for ch in range(3):
                pltpu.make_async_copy(
                    o_vs2[b][ch], out_hbm.at[pl.ds(ch * P, CH)], sem_out[b]).wait()

        fire_stage(0, 0)
        fire_stage(1, 1)

        @pl.loop(0, iters, step=2)
        def _(t):
          for b in range(2):
            g = t + b
            uv_v = uv_vs[b]
            idc_v = idc_vs[b]
            o_vs = o_vs2[b]
            base = wid * per_w + g * CH
            wait_stage(b)

            @pl.when(g >= 2)
            def _():
                wait_out(b)

            @pl.loop(0, CH // 128)
            def _(s):
                @pl.loop(0, 128 // L)
                def _(c):
                    off = s * 128 + c * L
                    u = plsc.load_gather(uv_v, [(off + lane) * 2])
                    v = plsc.load_gather(uv_v, [(off + lane) * 2 + 1])
                    m = idc_v[pl.ds(off, L)] == 0
                    halfw = jnp.where(m, (W0 - 1) * 0.5, (W1 - 1) * 0.5)
                    halfh = jnp.where(m, (H0 - 1) * 0.5, (H1 - 1) * 0.5)
                    maxx = jnp.where(m, float(W0 - 1), float(W1 - 1))
                    maxy = jnp.where(m, float(H0 - 1), float(H1 - 1))
                    wi = jnp.where(m, W0, W1)
                    bs = jnp.where(m, 0, base1)
                    x = jnp.clip((u + 1.0) * halfw, 0.0, maxx)
                    y = jnp.clip((v + 1.0) * halfh, 0.0, maxy)
                    xi = x.astype(jnp.int32)
                    yi = y.astype(jnp.int32)
                    wx1 = x - xi.astype(jnp.float32)
                    wy1 = y - yi.astype(jnp.float32)
                    x1 = jnp.minimum(xi + 1, maxx.astype(jnp.int32))
                    y1 = jnp.minimum(yi + 1, maxy.astype(jnp.int32))
                    r0 = bs + yi * wi
                    r1 = bs + y1 * wi
                    cs = pl.ds(c * L, L)
                    ls = pl.ds(off, L)
                    t00 = r0 + xi
                    t01 = r0 + x1
                    t10 = r1 + xi
                    t11 = r1 + x1
                    r_vs[0][s, cs] = t00 >> 1
                    r_vs[1][s, cs] = t01 >> 1
                    r_vs[2][s, cs] = t10 >> 1
                    r_vs[3][s, cs] = t11 >> 1
                    s_vs[0][ls] = (t00 & 1) << 2
                    s_vs[1][ls] = (t01 & 1) << 2
                    s_vs[2][ls] = (t10 & 1) << 2
                    s_vs[3][ls] = (t11 & 1) << 2
                    w_vs[0][ls] = (1.0 - wy1) * (1.0 - wx1)
                    w_vs[1][ls] = (1.0 - wy1) * wx1
                    w_vs[2][ls] = wy1 * (1.0 - wx1)
                    w_vs[3][ls] = wy1 * wx1

                for cn in range(4):
                    pltpu.async_copy(t_hbm.at[r_vs[cn].at[s]],
                                     g_vs[cn].at[pl.ds(s * 128, 128)], sem)

            for cn in range(4):
                for s in range(CH // 128):
                    pltpu.make_async_copy(
                        t_hbm.at[r_vs[cn].at[0]],
                        g_vs[cn].at[pl.ds(s * 128, 128)], sem).wait()

            @pl.loop(0, CH // L)
            def _(j):
                off = j * L
                pix = off + lane
                subs = [s_vs[cn][pl.ds(off, L)] for cn in range(4)]
                ws = [w_vs[cn][pl.ds(off, L)] for cn in range(4)]
                for ch in range(3):
                    acc = plsc.load_gather(g_vs[0], [pix, subs[0] + ch]) * ws[0]
                    for cn in range(1, 4):
                        acc = acc + plsc.load_gather(g_vs[cn], [pix, subs[cn] + ch]) * ws[cn]
                    o_vs[ch][pl.ds(off, L)] = acc

            for ch in range(3):
                pltpu.async_copy(o_vs[ch], out_hbm.at[pl.ds(ch * P + base, CH)],
                                 sem_out[b])

            @pl.when(g + 2 < iters)
            def _():
                fire_stage(g + 2, b)

        for b in range(2):
            wait_out(b)

    return kern(table, uv_flat, idc_flat)


def kernel(uv_coords, uv_idcs, subject_id, map0, map1):
    N, H, W, _ = uv_coords.shape
    M, C, H0, W0 = map0.shape
    H1, W1 = map1.shape[2], map1.shape[3]
    T0, T1 = H0 * W0, H1 * W1
    sid = subject_id[0]
    table = _sc_build_table(map0[sid].reshape(-1), map1[sid].reshape(-1), T0, T1)
    out = _sc_sample(table, uv_coords.reshape(-1), uv_idcs.reshape(-1).astype(jnp.int32),
                     H0, W0, H1, W1)
    return out.reshape(N, C, H, W)
